# Initial kernel scaffold; baseline (speedup 1.0000x reference)
import jax
import jax.numpy as jnp
from jax.experimental import pallas as pl


def kernel(x, edge_index, edge_attr, params):
    raise NotImplementedError("write your pallas kernel here")


# speedup vs baseline: 16.7994x; 16.7994x over previous
import functools

import jax
import jax.numpy as jnp
from jax import lax
from jax.experimental import pallas as pl
from jax.experimental.pallas import tpu as pltpu
from jax.experimental.pallas import tpu_sc as plsc

NP = 256
NN = 210
E_TOTAL = 6720
EDGES_PER_TILE = 224
TILES_PER_CORE = 15
FLAT = NP * NP


def _sc_edge_count_body(row_hbm, col_hbm, zero_hbm, out_hbm,
                        idx_v, idx_t_v, row_v, col_v, ones_v, acc_sh):
    cid = lax.axis_index("c")
    sid = lax.axis_index("s")

    @pl.when(sid == 0)
    def _():
        pltpu.sync_copy(zero_hbm, acc_sh)

    plsc.subcore_barrier()

    @pl.when(sid < TILES_PER_CORE)
    def _():
        base = (cid * TILES_PER_CORE + sid) * EDGES_PER_TILE
        pltpu.sync_copy(row_hbm.at[pl.ds(base, EDGES_PER_TILE)], row_v)
        pltpu.sync_copy(col_hbm.at[pl.ds(base, EDGES_PER_TILE)], col_v)

        def fill(j, carry):
            r = row_v[pl.ds(j * 16, 16)]
            c = col_v[pl.ds(j * 16, 16)]
            idx_v[pl.ds(j * 16, 16)] = c * NP + r
            idx_t_v[pl.ds(j * 16, 16)] = FLAT + r * NP + c
            ones_v[pl.ds(j * 16, 16)] = jnp.ones((16,), jnp.float32)
            return carry

        lax.fori_loop(0, EDGES_PER_TILE // 16, fill, 0)
        pltpu.sync_copy(ones_v, acc_sh.at[idx_v], add=True)
        pltpu.sync_copy(ones_v, acc_sh.at[idx_t_v], add=True)

    plsc.subcore_barrier()

    @pl.when(sid == 0)
    def _():
        pltpu.sync_copy(acc_sh, out_hbm.at[cid])


def _edge_counts_sc(row, col, zeros):
    mesh = plsc.VectorSubcoreMesh(core_axis_name="c", subcore_axis_name="s")
    k = pl.kernel(
        _sc_edge_count_body,
        out_type=jax.ShapeDtypeStruct((2, 2 * FLAT), jnp.float32),
        mesh=mesh,
        scratch_types=[
            pltpu.VMEM((EDGES_PER_TILE,), jnp.int32),
            pltpu.VMEM((EDGES_PER_TILE,), jnp.int32),
            pltpu.VMEM((EDGES_PER_TILE,), jnp.int32),
            pltpu.VMEM((EDGES_PER_TILE,), jnp.int32),
            pltpu.VMEM((EDGES_PER_TILE,), jnp.float32),
            pltpu.VMEM_SHARED((2 * FLAT,), jnp.float32),
        ],
    )
    return k(row, col, zeros)


def _build_l_body(cnt_ref, l_ref):
    a = cnt_ref[0, 0] + cnt_ref[1, 0]
    at = cnt_ref[0, 1] + cnt_ref[1, 1]
    deg_r = jnp.sum(a, axis=0, keepdims=True)
    deg_c = jnp.sum(at, axis=1, keepdims=True)
    dis_r = jnp.where(deg_r > 0, lax.rsqrt(jnp.where(deg_r > 0, deg_r, 1.0)), 0.0)
    dis_c = jnp.where(deg_c > 0, lax.rsqrt(jnp.where(deg_c > 0, deg_c, 1.0)), 0.0)
    l_ref[...] = -(a * dis_r) * dis_c


def _build_l(cnt):
    return pl.pallas_call(
        _build_l_body,
        out_shape=jax.ShapeDtypeStruct((NP, NP), jnp.float32),
    )(cnt.reshape(2, 2, NP, NP))


def _tconv_body(x_ref, w_ref, bias_ref, csum_ref, aff_ref,
                o_ref, stats_ref, accq_ref, accr_ref, *, k):
    t = pl.program_id(1)
    for d in range(k):
        xd = x_ref[0, t + d]
        p = jnp.dot(xd, w_ref[0, d], preferred_element_type=jnp.float32)
        q = jnp.dot(xd, w_ref[1, d], preferred_element_type=jnp.float32)
        r = jnp.dot(xd, w_ref[2, d], preferred_element_type=jnp.float32)
        if d == 0:
            o_ref[0, 0] = p
            accq_ref[...] = q
            accr_ref[...] = r
        else:
            o_ref[0, 0] += p
            accq_ref[...] += q
            accr_ref[...] += r
    scale = aff_ref[:, 0:1]
    shift = aff_ref[:, 1:2]
    pv = scale * o_ref[0, 0] + shift * csum_ref[0] + bias_ref[0]
    qv = scale * accq_ref[...] + shift * csum_ref[1] + bias_ref[1]
    rv = scale * accr_ref[...] + shift * csum_ref[2] + bias_ref[2]
    qv = 1.0 / (1.0 + jnp.exp(-qv))
    h = jnp.maximum(pv * qv + rv, 0.0)
    o_ref[0, 0] = h

    @pl.when((pl.program_id(0) == 0) & (t == 0))
    def _():
        stats_ref[...] = jnp.zeros_like(stats_ref)

    stats_ref[:, 0:1] += jnp.sum(h, axis=1, keepdims=True)
    stats_ref[:, 1:2] += jnp.sum(h * h, axis=1, keepdims=True)


def _tconv(x, w, bias, csum, aff, k):
    b, t_in, _, c_in = x.shape
    c_out = w.shape[-1]
    t_out = t_in - k + 1
    out = pl.pallas_call(
        functools.partial(_tconv_body, k=k),
        grid=(b, t_out),
        in_specs=[
            pl.BlockSpec((1, t_in, NP, c_in), lambda i, j: (i, 0, 0, 0)),
            pl.BlockSpec((3, k, c_in, c_out), lambda i, j: (0, 0, 0, 0)),
            pl.BlockSpec((3, 1, c_out), lambda i, j: (0, 0, 0)),
            pl.BlockSpec((3, 1, c_out), lambda i, j: (0, 0, 0)),
            pl.BlockSpec((NP, 2), lambda i, j: (0, 0)),
        ],
        out_specs=[
            pl.BlockSpec((1, 1, NP, c_out), lambda i, j: (i, j, 0, 0)),
            pl.BlockSpec((NP, 2), lambda i, j: (0, 0)),
        ],
        out_shape=[
            jax.ShapeDtypeStruct((b, t_out, NP, c_out), jnp.float32),
            jax.ShapeDtypeStruct((NP, 2), jnp.float32),
        ],
        scratch_shapes=[
            pltpu.VMEM((NP, c_out), jnp.float32),
            pltpu.VMEM((NP, c_out), jnp.float32),
        ],
    )(x, w, bias, csum, aff)
    return out


def _cheb_body(h_ref, l_ref, w0_ref, w1_ref, w2_ref, bias_ref, o_ref):
    x = h_ref[0, 0]
    lm = l_ref[...]
    tx1 = jnp.dot(lm, x, preferred_element_type=jnp.float32)
    tx2 = 2.0 * jnp.dot(lm, tx1, preferred_element_type=jnp.float32) - x
    out = (jnp.dot(x, w0_ref[...], preferred_element_type=jnp.float32)
           + jnp.dot(tx1, w1_ref[...], preferred_element_type=jnp.float32)
           + jnp.dot(tx2, w2_ref[...], preferred_element_type=jnp.float32)
           + bias_ref[...])
    o_ref[0, 0] = jnp.maximum(out, 0.0)


def _cheb(h, lm, w0t, w1t, w2t, bias):
    b, t, _, c = h.shape
    return pl.pallas_call(
        _cheb_body,
        grid=(b, t),
        in_specs=[
            pl.BlockSpec((1, 1, NP, c), lambda i, j: (i, j, 0, 0)),
            pl.BlockSpec((NP, NP), lambda i, j: (0, 0)),
            pl.BlockSpec((c, c), lambda i, j: (0, 0)),
            pl.BlockSpec((c, c), lambda i, j: (0, 0)),
            pl.BlockSpec((c, c), lambda i, j: (0, 0)),
            pl.BlockSpec((1, c), lambda i, j: (0, 0)),
        ],
        out_specs=pl.BlockSpec((1, 1, NP, c), lambda i, j: (i, j, 0, 0)),
        out_shape=jax.ShapeDtypeStruct((b, t, NP, c), jnp.float32),
    )(h, lm, w0t, w1t, w2t, bias)


def _bn_apply_body(x_ref, aff_ref, o_ref):
    scale = aff_ref[:, 0:1][None, None]
    shift = aff_ref[:, 1:2][None, None]
    o_ref[...] = x_ref[...] * scale + shift


def _bn_apply(x, aff):
    b, t, _, c = x.shape
    return pl.pallas_call(
        _bn_apply_body,
        grid=(b,),
        in_specs=[
            pl.BlockSpec((1, t, NP, c), lambda i: (i, 0, 0, 0)),
            pl.BlockSpec((NP, 2), lambda i: (0, 0)),
        ],
        out_specs=pl.BlockSpec((1, t, NP, c), lambda i: (i, 0, 0, 0)),
        out_shape=jax.ShapeDtypeStruct(x.shape, jnp.float32),
    )(x, aff)


def _prep_tconv(p, pre):
    ws = jnp.stack([jnp.transpose(p[pre + 'w1'][:, :, 0, :], (2, 1, 0)),
                    jnp.transpose(p[pre + 'w2'][:, :, 0, :], (2, 1, 0)),
                    jnp.transpose(p[pre + 'w3'][:, :, 0, :], (2, 1, 0))])
    bias = jnp.stack([p[pre + 'b1'][None, :], p[pre + 'b2'][None, :],
                      p[pre + 'b3'][None, :]])
    csum = jnp.sum(ws, axis=(1, 2), keepdims=False)[:, None, :]
    return ws, bias, csum


def _bn_affine(stats, gamma, beta, count):
    s1 = stats[:, 0]
    s2 = stats[:, 1]
    mean = s1 / count
    var = s2 / count - mean * mean
    scale = gamma / jnp.sqrt(var + 1e-5)
    shift = beta - mean * scale
    return jnp.stack([scale, shift], axis=1)


def _identity_aff():
    return jnp.stack([jnp.ones((NP,), jnp.float32),
                      jnp.zeros((NP,), jnp.float32)], axis=1)


def kernel(x, edge_index, edge_attr, params):
    del edge_attr
    b, t, _, _ = x.shape

    row = edge_index[0].astype(jnp.int32)
    col = edge_index[1].astype(jnp.int32)
    cnt = _edge_counts_sc(row, col, jnp.zeros((2 * FLAT,), jnp.float32))
    lm = _build_l(cnt)

    xp = jnp.pad(x, ((0, 0), (0, 0), (0, NP - NN), (0, 0)))

    aff = _identity_aff()
    ident = _identity_aff()
    ks = {'b1': 9, 'b2': 7, 'b3': 3}
    h = xp
    for name in ('b1', 'b2', 'b3'):
        p = params[name]
        k = ks[name]
        w1s, b1s, c1s = _prep_tconv(p, 't1_')
        w2s, b2s, c2s = _prep_tconv(p, 't2_')
        h, _ = _tconv(h, w1s, b1s, c1s, aff, k)
        h = _cheb(h, lm, p['cw0'].T, p['cw1'].T, p['cw2'].T, p['cb'][None, :])
        h, stats = _tconv(h, w2s, b2s, c2s, ident, k)
        count = h.shape[0] * h.shape[1] * h.shape[3]
        gamma = jnp.pad(p['bn_g'], (0, NP - NN), constant_values=1.0)
        beta = jnp.pad(p['bn_b'], (0, NP - NN))
        aff = _bn_affine(stats, gamma, beta, float(count))

    out = _bn_apply(h, aff)
    return out[:, :, :NN, :]
